# baseline (speedup 1.0000x reference)
import jax
import jax.numpy as jnp
from jax import lax
from jax.experimental import pallas as pl
from jax.experimental.pallas import tpu as pltpu
from jax.experimental.pallas import tpu_sc as plsc

D_MODEL = 128
MAX_REL = 128
LENGTH = 1024
BASE = 1152
B_ROWS = 2176
NWORKERS = 32
ROWS_PER_TILE = LENGTH // NWORKERS
NSEM = 4


def _sc_body(table_hbm, out_hbm, rep0, rep1, b_sh, sems):
    c = lax.axis_index("c")
    s = lax.axis_index("s")
    wid = s * 2 + c

    @pl.when(s == 0)
    def _fill():
        for rep, src_row, dst0 in ((rep0, 0, 0), (rep1, 256, 1280)):
            pltpu.sync_copy(
                table_hbm.at[pl.ds(src_row, 1)], rep.at[pl.ds(0, 1)]
            )
            pltpu.sync_copy(rep.at[pl.ds(0, 1)], b_sh.at[pl.ds(dst0, 1)])
            n = 1
            while n < 128:
                pltpu.sync_copy(
                    b_sh.at[pl.ds(dst0, n)], rep.at[pl.ds(n, n)]
                )
                pltpu.sync_copy(
                    rep.at[pl.ds(n, n)], b_sh.at[pl.ds(dst0 + n, n)]
                )
                n *= 2
            nblocks = 8 if dst0 == 0 else 7
            for k in range(1, nblocks):
                pltpu.sync_copy(rep, b_sh.at[pl.ds(dst0 + 128 * k, 128)])
        pltpu.sync_copy(
            table_hbm.at[pl.ds(0, 256)], b_sh.at[pl.ds(1024, 256)]
        )

    plsc.subcore_barrier()

    i0 = wid * ROWS_PER_TILE

    def mk(t):
        return pltpu.make_async_copy(
            b_sh.at[pl.ds(BASE - (i0 + t), LENGTH)],
            out_hbm.at[i0 + t],
            sems.at[t % NSEM],
        )

    for t in range(ROWS_PER_TILE):
        mk(t).start()
        if t >= NSEM - 1:
            mk(t - (NSEM - 1)).wait()
    for t in range(ROWS_PER_TILE - (NSEM - 1), ROWS_PER_TILE):
        mk(t).wait()


def kernel(length, table):
    del length
    mesh = plsc.VectorSubcoreMesh(core_axis_name="c", subcore_axis_name="s")
    run = pl.kernel(
        _sc_body,
        mesh=mesh,
        out_type=jax.ShapeDtypeStruct((LENGTH, LENGTH, D_MODEL), jnp.float32),
        scratch_types=[
            pltpu.VMEM((128, D_MODEL), jnp.float32),
            pltpu.VMEM((128, D_MODEL), jnp.float32),
            pltpu.VMEM_SHARED((B_ROWS, D_MODEL), jnp.float32),
            pltpu.SemaphoreType.DMA((NSEM,)),
        ],
    )
    return run(table)


# speedup vs baseline: 1.3693x; 1.3693x over previous
import jax
import jax.numpy as jnp
from jax import lax
from jax.experimental import pallas as pl
from jax.experimental.pallas import tpu as pltpu
from jax.experimental.pallas import tpu_sc as plsc

D_MODEL = 128
MAX_REL = 128
LENGTH = 1024
BASE = 1152
B_ROWS = 2176
B_ROWS_PAD = 2304
NWORKERS = 32
ROWS_PER_TILE = B_ROWS_PAD // NWORKERS
ROWS_PER_STEP = 8
LANES = 16


def _sc_gather_body(table_hbm, b_hbm, idx_v, rows_v, sem):
    c = lax.axis_index("c")
    s = lax.axis_index("s")
    wid = s * 2 + c
    base = wid * ROWS_PER_TILE

    for o in (0, 16, 32, 48, ROWS_PER_TILE - LANES):
        e = base + o + lax.iota(jnp.int32, LANES)
        idx = jnp.clip(e - BASE, -MAX_REL, MAX_REL) + MAX_REL
        idx_v[pl.ds(o, LANES)] = idx

    pltpu.async_copy(table_hbm.at[idx_v], rows_v, sem).wait()
    pltpu.sync_copy(rows_v, b_hbm.at[pl.ds(base, ROWS_PER_TILE)])


def _tc_stream_body(b_ref, out_ref):
    i0 = pl.program_id(0) * ROWS_PER_STEP
    for r in range(ROWS_PER_STEP):
        out_ref[r, :, :] = b_ref[pl.ds(BASE - (i0 + r), LENGTH), :]


def kernel(length, table):
    del length
    mesh = plsc.VectorSubcoreMesh(core_axis_name="c", subcore_axis_name="s")
    b = pl.kernel(
        _sc_gather_body,
        mesh=mesh,
        out_type=jax.ShapeDtypeStruct((B_ROWS_PAD, D_MODEL), jnp.float32),
        scratch_types=[
            pltpu.VMEM((ROWS_PER_TILE,), jnp.int32),
            pltpu.VMEM((ROWS_PER_TILE, D_MODEL), jnp.float32),
            pltpu.SemaphoreType.DMA,
        ],
    )(table)
    return pl.pallas_call(
        _tc_stream_body,
        grid=(LENGTH // ROWS_PER_STEP,),
        in_specs=[pl.BlockSpec((B_ROWS_PAD, D_MODEL), lambda i: (0, 0))],
        out_specs=pl.BlockSpec(
            (ROWS_PER_STEP, LENGTH, D_MODEL), lambda i: (i, 0, 0)
        ),
        out_shape=jax.ShapeDtypeStruct((LENGTH, LENGTH, D_MODEL), jnp.float32),
    )(b)
